# Initial kernel scaffold; baseline (speedup 1.0000x reference)
import functools

import jax
import jax.numpy as jnp
from jax import lax
from jax.experimental import pallas as pl
from jax.experimental.pallas import tpu as pltpu
from jax.experimental.pallas import tpu_sc as plsc

_N = 10000
_E = 320000
_D = 128
_NC = 2
_NS = 16
_NW = _NC * _NS
_L = 16

_NP = 10240
_EP = 327680
_EW = _EP // 128
_WPW = _EW // _NW
_WPT = _EW // _NS
_RPT = _NP // _NS

_mesh = plsc.VectorSubcoreMesh(core_axis_name="c", subcore_axis_name="s")


def _z16():
    return jnp.zeros((_L,), jnp.float32)


def _norm_body(row2, col2, ea2, norm2,
               deg_sp, dinv_sp, colv, eav, zv, degv, dinvv, dinvf,
               rowc, colc, eac, normc):
    c = lax.axis_index("c")
    s = lax.axis_index("s")
    wid = s * _NC + c

    @pl.loop(0, _RPT // _L)
    def _p1(i):
        zv[pl.ds(i * _L, _L)] = _z16()

    pltpu.sync_copy(zv, deg_sp.at[pl.ds(s * _RPT, _RPT)])
    plsc.subcore_barrier()

    pltpu.sync_copy(col2.at[pl.ds(s * _WPT, _WPT)], colv)
    pltpu.sync_copy(ea2.at[pl.ds(s * _WPT, _WPT)], eav)

    @pl.loop(0, _WPT)
    def _p2(w):
        pltpu.sync_copy(eav.at[w], deg_sp.at[colv.at[w]], add=True)

    plsc.subcore_barrier()

    pltpu.sync_copy(deg_sp.at[pl.ds(s * _RPT, _RPT)], degv)

    @pl.loop(0, _RPT // _L)
    def _p3(i):
        sl = pl.ds(i * _L, _L)
        d = degv[sl]
        bits = plsc.bitcast(d, jnp.int32)
        y = plsc.bitcast(jnp.int32(0x5F3759DF) - (bits >> 1), jnp.float32)
        for _ in range(3):
            y = y * (1.5 - 0.5 * d * y * y)
        dinvv[sl] = jnp.where(d > 0.0, y, 0.0)

    pltpu.sync_copy(dinvv, dinv_sp.at[pl.ds(s * _RPT, _RPT)])
    plsc.subcore_barrier()

    pltpu.sync_copy(dinv_sp, dinvf)
    base = wid * _WPW
    pltpu.sync_copy(row2.at[pl.ds(base, _WPW)], rowc)
    pltpu.sync_copy(col2.at[pl.ds(base, _WPW)], colc)
    pltpu.sync_copy(ea2.at[pl.ds(base, _WPW)], eac)

    @pl.loop(0, _WPW)
    def _p4(w):
        for j in range(8):
            sl = pl.ds(j * _L, _L)
            dr = plsc.load_gather(dinvf, [rowc[w, sl]])
            dc = plsc.load_gather(dinvf, [colc[w, sl]])
            normc[w, sl] = dr * eac[w, sl] * dc

    pltpu.sync_copy(normc, norm2.at[pl.ds(base, _WPW)])


@jax.jit
def _norm_sc(row2, col2, ea2):
    f = pl.kernel(
        _norm_body,
        out_type=jax.ShapeDtypeStruct((_EW, 128), jnp.float32),
        mesh=_mesh,
        scratch_types=[
            pltpu.VMEM_SHARED((_NP,), jnp.float32),
            pltpu.VMEM_SHARED((_NP,), jnp.float32),
            pltpu.VMEM((_WPT, 128), jnp.int32),
            pltpu.VMEM((_WPT, 128), jnp.float32),
            pltpu.VMEM((_RPT,), jnp.float32),
            pltpu.VMEM((_RPT,), jnp.float32),
            pltpu.VMEM((_RPT,), jnp.float32),
            pltpu.VMEM((_NP,), jnp.float32),
            pltpu.VMEM((_WPW, 128), jnp.int32),
            pltpu.VMEM((_WPW, 128), jnp.int32),
            pltpu.VMEM((_WPW, 128), jnp.float32),
            pltpu.VMEM((_WPW, 128), jnp.float32),
        ],
    )
    return f(row2, col2, ea2)


def _prop_body(h, row2, col2, norm2, out0, out1,
               acc_sp, rowc, colc, normc, zb, msg):
    c = lax.axis_index("c")
    s = lax.axis_index("s")
    wid = s * _NC + c

    @pl.loop(0, 128)
    def _pz(i):
        for j in range(8):
            zb[i, pl.ds(j * _L, _L)] = _z16()

    @pl.loop(0, _RPT // 128)
    def _pz2(i):
        pltpu.sync_copy(zb, acc_sp.at[pl.ds(s * _RPT + i * 128, 128)])

    plsc.subcore_barrier()

    base = wid * _WPW
    pltpu.sync_copy(row2.at[pl.ds(base, _WPW)], rowc)
    pltpu.sync_copy(col2.at[pl.ds(base, _WPW)], colc)
    pltpu.sync_copy(norm2.at[pl.ds(base, _WPW)], normc)

    @pl.loop(0, _WPW)
    def _pw(w):
        pltpu.sync_copy(h.at[rowc.at[w]], msg)

        @pl.loop(0, 128)
        def _pe(e):
            nb = plsc.load_gather(
                normc, [jnp.full((_L,), w, jnp.int32),
                        jnp.full((_L,), e, jnp.int32)])
            for j in range(8):
                sl = pl.ds(j * _L, _L)
                msg[e, sl] = msg[e, sl] * nb

        pltpu.sync_copy(msg, acc_sp.at[colc.at[w]], add=True)

    plsc.subcore_barrier()

    @pl.when(c == 0)
    def _w0():
        pltpu.sync_copy(acc_sp.at[pl.ds(s * _RPT, _RPT)],
                        out0.at[pl.ds(s * _RPT, _RPT)])

    @pl.when(c == 1)
    def _w1():
        pltpu.sync_copy(acc_sp.at[pl.ds(s * _RPT, _RPT)],
                        out1.at[pl.ds(s * _RPT, _RPT)])


@jax.jit
def _prop_sc(h, row2, col2, norm2):
    f = pl.kernel(
        _prop_body,
        out_type=[jax.ShapeDtypeStruct((_NP, _D), jnp.float32),
                  jax.ShapeDtypeStruct((_NP, _D), jnp.float32)],
        mesh=_mesh,
        scratch_types=[
            pltpu.VMEM_SHARED((_NP, _D), jnp.float32),
            pltpu.VMEM((_WPW, 128), jnp.int32),
            pltpu.VMEM((_WPW, 128), jnp.int32),
            pltpu.VMEM((_WPW, 128), jnp.float32),
            pltpu.VMEM((128, _D), jnp.float32),
            pltpu.VMEM((128, _D), jnp.float32),
        ],
    )
    return f(h, row2, col2, norm2)


def _combine_body(a_ref, b_ref, o_ref):
    o_ref[...] = a_ref[...] + b_ref[...]


_combine = pl.pallas_call(
    _combine_body,
    grid=(8,),
    in_specs=[pl.BlockSpec((_NP // 8, _D), lambda i: (i, 0))] * 2,
    out_specs=pl.BlockSpec((_NP // 8, _D), lambda i: (i, 0)),
    out_shape=jax.ShapeDtypeStruct((_NP, _D), jnp.float32),
)


def _layer_body(x_ref, h1_ref, h2_ref, p0_ref, p1_ref, w_ref, b_ref, o_ref):
    acc = jnp.dot(x_ref[...], w_ref[0], preferred_element_type=jnp.float32)
    acc = acc + jnp.dot(h1_ref[...], w_ref[1], preferred_element_type=jnp.float32)
    acc = acc + jnp.dot(h2_ref[...], w_ref[2], preferred_element_type=jnp.float32)
    h3 = p0_ref[...] + p1_ref[...]
    acc = acc + jnp.dot(h3, w_ref[3], preferred_element_type=jnp.float32)
    acc = acc + b_ref[...]
    o_ref[...] = jnp.where(acc > 0, acc, 0.01 * acc)


_layer = pl.pallas_call(
    _layer_body,
    grid=(8,),
    in_specs=[pl.BlockSpec((_NP // 8, _D), lambda i: (i, 0))] * 5
    + [pl.BlockSpec((4, _D, _D), lambda i: (0, 0, 0)),
       pl.BlockSpec((1, _D), lambda i: (0, 0))],
    out_specs=pl.BlockSpec((_NP // 8, _D), lambda i: (i, 0)),
    out_shape=jax.ShapeDtypeStruct((_NP, _D), jnp.float32),
)


def kernel(y, edge_index, edge_attr, W1, b1, W2):
    row = edge_index[0]
    col = edge_index[1]
    pad = _EP - _E
    row2 = jnp.pad(row, (0, pad)).reshape(_EW, 128)
    col2 = jnp.pad(col, (0, pad)).reshape(_EW, 128)
    ea2 = jnp.pad(edge_attr, (0, pad)).reshape(_EW, 128)
    x = jnp.pad(y, ((0, _NP - _N), (0, 0)))

    norm2 = _norm_sc(row2, col2, ea2)
    b1r = b1.reshape(1, _D)
    zb = jnp.zeros((1, _D), jnp.float32)

    for W, b in ((W1, b1r), (W2, zb)):
        p10, p11 = _prop_sc(x, row2, col2, norm2)
        h1 = _combine(p10, p11)
        p20, p21 = _prop_sc(h1, row2, col2, norm2)
        h2 = _combine(p20, p21)
        p30, p31 = _prop_sc(h2, row2, col2, norm2)
        x = _layer(x, h1, h2, p30, p31, W, b)
    return x[:_N]


# speedup vs baseline: 3.6156x; 3.6156x over previous
import dataclasses
import functools

import jax
import jax.numpy as jnp
from jax import lax
from jax.experimental import pallas as pl
from jax.experimental.pallas import tpu as pltpu
from jax.experimental.pallas import tpu_sc as plsc

_N = 10000
_E = 320000
_D = 128
_NC = 2
_NS = 16
_NW = _NC * _NS
_L = 16

_NP = 10240
_EP = 327680
_EW = _EP // 128
_WPW = _EW // _NW
_WPT = _EW // _NS
_RPT = _NP // _NS

_mesh = plsc.VectorSubcoreMesh(core_axis_name="c", subcore_axis_name="s")

_sc_params = pltpu.CompilerParams()
if "needs_layout_passes" in pltpu.CompilerParams.__dataclass_fields__:
    _sc_params = dataclasses.replace(_sc_params, needs_layout_passes=False)


def _z16():
    return jnp.zeros((_L,), jnp.float32)


def _norm_body(row2, col2, ea2, norm2,
               deg_sp, dinv_sp, colv, eav, zv, degv, dinvv, dinvf,
               rowc, colc, eac, normc):
    c = lax.axis_index("c")
    s = lax.axis_index("s")
    wid = s * _NC + c

    @pl.loop(0, _RPT // _L)
    def _p1(i):
        zv[pl.ds(i * _L, _L)] = _z16()

    pltpu.sync_copy(zv, deg_sp.at[pl.ds(s * _RPT, _RPT)])
    plsc.subcore_barrier()

    pltpu.sync_copy(col2.at[pl.ds(s * _WPT, _WPT)], colv)
    pltpu.sync_copy(ea2.at[pl.ds(s * _WPT, _WPT)], eav)

    @pl.loop(0, _WPT)
    def _p2(w):
        pltpu.sync_copy(eav.at[w], deg_sp.at[colv.at[w]], add=True)

    plsc.subcore_barrier()

    pltpu.sync_copy(deg_sp.at[pl.ds(s * _RPT, _RPT)], degv)

    @pl.loop(0, _RPT // _L)
    def _p3(i):
        sl = pl.ds(i * _L, _L)
        d = degv[sl]
        bits = plsc.bitcast(d, jnp.int32)
        y = plsc.bitcast(jnp.int32(0x5F3759DF) - (bits >> 1), jnp.float32)
        for _ in range(3):
            y = y * (1.5 - 0.5 * d * y * y)
        dinvv[sl] = jnp.where(d > 0.0, y, 0.0)

    pltpu.sync_copy(dinvv, dinv_sp.at[pl.ds(s * _RPT, _RPT)])
    plsc.subcore_barrier()

    pltpu.sync_copy(dinv_sp, dinvf)
    base = wid * _WPW
    pltpu.sync_copy(row2.at[pl.ds(base, _WPW)], rowc)
    pltpu.sync_copy(col2.at[pl.ds(base, _WPW)], colc)
    pltpu.sync_copy(ea2.at[pl.ds(base, _WPW)], eac)

    @pl.loop(0, _WPW)
    def _p4(w):
        for j in range(8):
            sl = pl.ds(j * _L, _L)
            dr = plsc.load_gather(dinvf, [rowc[w, sl]])
            dc = plsc.load_gather(dinvf, [colc[w, sl]])
            normc[w, sl] = dr * eac[w, sl] * dc

    pltpu.sync_copy(normc, norm2.at[pl.ds(base, _WPW)])


@jax.jit
def _norm_sc(row2, col2, ea2):
    f = pl.kernel(
        _norm_body,
        out_type=jax.ShapeDtypeStruct((_EW, 128), jnp.float32),
        mesh=_mesh,
        compiler_params=_sc_params,
        scratch_types=[
            pltpu.VMEM_SHARED((_NP,), jnp.float32),
            pltpu.VMEM_SHARED((_NP,), jnp.float32),
            pltpu.VMEM((_WPT, 128), jnp.int32),
            pltpu.VMEM((_WPT, 128), jnp.float32),
            pltpu.VMEM((_RPT,), jnp.float32),
            pltpu.VMEM((_RPT,), jnp.float32),
            pltpu.VMEM((_RPT,), jnp.float32),
            pltpu.VMEM((_NP,), jnp.float32),
            pltpu.VMEM((_WPW, 128), jnp.int32),
            pltpu.VMEM((_WPW, 128), jnp.int32),
            pltpu.VMEM((_WPW, 128), jnp.float32),
            pltpu.VMEM((_WPW, 128), jnp.float32),
        ],
    )
    return f(row2, col2, ea2)


_CW = 16


def _prop_body(h, row2, col2, norm2, out0, out1,
               acc_sp, rowc, colc, normc, zb, msg):
    c = lax.axis_index("c")
    s = lax.axis_index("s")
    wid = s * _NC + c

    @pl.loop(0, _CW)
    def _pz(i):
        for j in range(8):
            zb[i, pl.ds(j * _L, _L)] = _z16()

    @pl.loop(0, _RPT // _CW)
    def _pz2(i):
        pltpu.sync_copy(zb, acc_sp.at[pl.ds(s * _RPT + i * _CW, _CW)])

    plsc.subcore_barrier()

    base = wid * _WPW

    @pl.loop(0, _WPW // _CW)
    def _pc(ci):
        cbase = base + ci * _CW
        pltpu.sync_copy(row2.at[pl.ds(cbase, _CW)], rowc)
        pltpu.sync_copy(col2.at[pl.ds(cbase, _CW)], colc)
        pltpu.sync_copy(norm2.at[pl.ds(cbase, _CW)], normc)

        @pl.loop(0, _CW)
        def _pw(w):
            pltpu.sync_copy(h.at[rowc.at[w]], msg)

            @pl.loop(0, 128)
            def _pe(e):
                nb = plsc.load_gather(
                    normc, [jnp.full((_L,), w, jnp.int32),
                            jnp.full((_L,), e, jnp.int32)])
                for j in range(8):
                    sl = pl.ds(j * _L, _L)
                    msg[e, sl] = msg[e, sl] * nb

            pltpu.sync_copy(msg, acc_sp.at[colc.at[w]], add=True)

    plsc.subcore_barrier()

    @pl.when(c == 0)
    def _w0():
        pltpu.sync_copy(acc_sp.at[pl.ds(s * _RPT, _RPT)],
                        out0.at[pl.ds(s * _RPT, _RPT)])

    @pl.when(c == 1)
    def _w1():
        pltpu.sync_copy(acc_sp.at[pl.ds(s * _RPT, _RPT)],
                        out1.at[pl.ds(s * _RPT, _RPT)])


@jax.jit
def _prop_sc(h, row2, col2, norm2):
    f = pl.kernel(
        _prop_body,
        out_type=[jax.ShapeDtypeStruct((_NP, _D), jnp.float32),
                  jax.ShapeDtypeStruct((_NP, _D), jnp.float32)],
        mesh=_mesh,
        compiler_params=_sc_params,
        scratch_types=[
            pltpu.VMEM_SHARED((_NP, _D), jnp.float32),
            pltpu.VMEM((_CW, 128), jnp.int32),
            pltpu.VMEM((_CW, 128), jnp.int32),
            pltpu.VMEM((_CW, 128), jnp.float32),
            pltpu.VMEM((_CW, _D), jnp.float32),
            pltpu.VMEM((128, _D), jnp.float32),
        ],
    )
    return f(h, row2, col2, norm2)


def _combine_body(a_ref, b_ref, o_ref):
    o_ref[...] = a_ref[...] + b_ref[...]


_combine = pl.pallas_call(
    _combine_body,
    grid=(8,),
    in_specs=[pl.BlockSpec((_NP // 8, _D), lambda i: (i, 0))] * 2,
    out_specs=pl.BlockSpec((_NP // 8, _D), lambda i: (i, 0)),
    out_shape=jax.ShapeDtypeStruct((_NP, _D), jnp.float32),
)


def _layer_body(x_ref, h1_ref, h2_ref, p0_ref, p1_ref, w_ref, b_ref, o_ref):
    acc = jnp.dot(x_ref[...], w_ref[0], preferred_element_type=jnp.float32)
    acc = acc + jnp.dot(h1_ref[...], w_ref[1], preferred_element_type=jnp.float32)
    acc = acc + jnp.dot(h2_ref[...], w_ref[2], preferred_element_type=jnp.float32)
    h3 = p0_ref[...] + p1_ref[...]
    acc = acc + jnp.dot(h3, w_ref[3], preferred_element_type=jnp.float32)
    acc = acc + b_ref[...]
    o_ref[...] = jnp.where(acc > 0, acc, 0.01 * acc)


_layer = pl.pallas_call(
    _layer_body,
    grid=(8,),
    in_specs=[pl.BlockSpec((_NP // 8, _D), lambda i: (i, 0))] * 5
    + [pl.BlockSpec((4, _D, _D), lambda i: (0, 0, 0)),
       pl.BlockSpec((1, _D), lambda i: (0, 0))],
    out_specs=pl.BlockSpec((_NP // 8, _D), lambda i: (i, 0)),
    out_shape=jax.ShapeDtypeStruct((_NP, _D), jnp.float32),
)


def kernel(y, edge_index, edge_attr, W1, b1, W2):
    row = edge_index[0]
    col = edge_index[1]
    pad = _EP - _E
    row2 = jnp.pad(row, (0, pad)).reshape(_EW, 128)
    col2 = jnp.pad(col, (0, pad)).reshape(_EW, 128)
    ea2 = jnp.pad(edge_attr, (0, pad)).reshape(_EW, 128)
    x = jnp.pad(y, ((0, _NP - _N), (0, 0)))

    norm2 = _norm_sc(row2, col2, ea2)
    b1r = b1.reshape(1, _D)
    zb = jnp.zeros((1, _D), jnp.float32)

    for W, b in ((W1, b1r), (W2, zb)):
        p10, p11 = _prop_sc(x, row2, col2, norm2)
        h1 = _combine(p10, p11)
        p20, p21 = _prop_sc(h1, row2, col2, norm2)
        h2 = _combine(p20, p21)
        p30, p31 = _prop_sc(h2, row2, col2, norm2)
        x = _layer(x, h1, h2, p30, p31, W, b)
    return x[:_N]


# speedup vs baseline: 4.3703x; 1.2087x over previous
import dataclasses
import functools

import jax
import jax.numpy as jnp
from jax import lax
from jax.experimental import pallas as pl
from jax.experimental.pallas import tpu as pltpu
from jax.experimental.pallas import tpu_sc as plsc

_N = 10000
_E = 320000
_D = 128
_NC = 2
_NS = 16
_NW = _NC * _NS
_L = 16

_NP = 10240
_EP = 327680
_EW = _EP // 128
_WPW = _EW // _NW
_WPT = _EW // _NS
_RPT = _NP // _NS

_mesh = plsc.VectorSubcoreMesh(core_axis_name="c", subcore_axis_name="s")

_sc_params = pltpu.CompilerParams()
if "needs_layout_passes" in pltpu.CompilerParams.__dataclass_fields__:
    _sc_params = dataclasses.replace(_sc_params, needs_layout_passes=False)


def _z16():
    return jnp.zeros((_L,), jnp.float32)


def _norm_body(row2, col2, ea2, norm2,
               deg_sp, dinv_sp, colv, eav, zv, degv, dinvv, dinvf,
               rowc, colc, eac, normc):
    c = lax.axis_index("c")
    s = lax.axis_index("s")
    wid = s * _NC + c

    @pl.loop(0, _RPT // _L)
    def _p1(i):
        zv[pl.ds(i * _L, _L)] = _z16()

    pltpu.sync_copy(zv, deg_sp.at[pl.ds(s * _RPT, _RPT)])
    plsc.subcore_barrier()

    pltpu.sync_copy(col2.at[pl.ds(s * _WPT, _WPT)], colv)
    pltpu.sync_copy(ea2.at[pl.ds(s * _WPT, _WPT)], eav)

    @pl.loop(0, _WPT)
    def _p2(w):
        pltpu.sync_copy(eav.at[w], deg_sp.at[colv.at[w]], add=True)

    plsc.subcore_barrier()

    pltpu.sync_copy(deg_sp.at[pl.ds(s * _RPT, _RPT)], degv)

    @pl.loop(0, _RPT // _L)
    def _p3(i):
        sl = pl.ds(i * _L, _L)
        d = degv[sl]
        bits = plsc.bitcast(d, jnp.int32)
        y = plsc.bitcast(jnp.int32(0x5F3759DF) - (bits >> 1), jnp.float32)
        for _ in range(3):
            y = y * (1.5 - 0.5 * d * y * y)
        dinvv[sl] = jnp.where(d > 0.0, y, 0.0)

    pltpu.sync_copy(dinvv, dinv_sp.at[pl.ds(s * _RPT, _RPT)])
    plsc.subcore_barrier()

    pltpu.sync_copy(dinv_sp, dinvf)
    base = wid * _WPW
    pltpu.sync_copy(row2.at[pl.ds(base, _WPW)], rowc)
    pltpu.sync_copy(col2.at[pl.ds(base, _WPW)], colc)
    pltpu.sync_copy(ea2.at[pl.ds(base, _WPW)], eac)

    @pl.loop(0, _WPW)
    def _p4(w):
        for j in range(8):
            sl = pl.ds(j * _L, _L)
            dr = plsc.load_gather(dinvf, [rowc[w, sl]])
            dc = plsc.load_gather(dinvf, [colc[w, sl]])
            normc[w, sl] = dr * eac[w, sl] * dc

    pltpu.sync_copy(normc, norm2.at[pl.ds(base, _WPW)])


@jax.jit
def _norm_sc(row2, col2, ea2):
    f = pl.kernel(
        _norm_body,
        out_type=jax.ShapeDtypeStruct((_EW, 128), jnp.float32),
        mesh=_mesh,
        compiler_params=_sc_params,
        scratch_types=[
            pltpu.VMEM_SHARED((_NP,), jnp.float32),
            pltpu.VMEM_SHARED((_NP,), jnp.float32),
            pltpu.VMEM((_WPT, 128), jnp.int32),
            pltpu.VMEM((_WPT, 128), jnp.float32),
            pltpu.VMEM((_RPT,), jnp.float32),
            pltpu.VMEM((_RPT,), jnp.float32),
            pltpu.VMEM((_RPT,), jnp.float32),
            pltpu.VMEM((_NP,), jnp.float32),
            pltpu.VMEM((_WPW, 128), jnp.int32),
            pltpu.VMEM((_WPW, 128), jnp.int32),
            pltpu.VMEM((_WPW, 128), jnp.float32),
            pltpu.VMEM((_WPW, 128), jnp.float32),
        ],
    )
    return f(row2, col2, ea2)


_WE = 64
_EW64 = _EP // _WE
_W64 = _EW64 // _NW
_CWIN = 40


def _prop_body(h, row2, col2, norm2, out0, out1,
               acc_sp, rowc, colc, normc, msg0, msg1, gsem0, gsem1, zsem):
    c = lax.axis_index("c")
    s = lax.axis_index("s")
    wid = s * _NC + c

    @pl.loop(0, _WE)
    def _pz(i):
        for j in range(8):
            msg0[i, pl.ds(j * _L, _L)] = _z16()

    @pl.loop(0, _RPT // _WE)
    def _pz2(i):
        pltpu.async_copy(msg0, acc_sp.at[pl.ds(s * _RPT + i * _WE, _WE)], zsem)

    @pl.loop(0, _RPT // _WE)
    def _pz3(i):
        pltpu.make_async_copy(msg0, acc_sp.at[pl.ds(s * _RPT, _WE)], zsem).wait()

    plsc.subcore_barrier()

    base = wid * _W64

    def scale_scatter(w, msg):
        @pl.loop(0, _WE)
        def _pe(e):
            nb = plsc.load_gather(
                normc, [jnp.full((_L,), w, jnp.int32),
                        jnp.full((_L,), e, jnp.int32)])
            for j in range(8):
                sl = pl.ds(j * _L, _L)
                msg[e, sl] = msg[e, sl] * nb

        pltpu.sync_copy(msg, acc_sp.at[colc.at[w]], add=True)

    @pl.loop(0, _W64 // _CWIN)
    def _pc(ci):
        cb = base + ci * _CWIN
        pltpu.sync_copy(row2.at[pl.ds(cb, _CWIN)], rowc)
        pltpu.sync_copy(col2.at[pl.ds(cb, _CWIN)], colc)
        pltpu.sync_copy(norm2.at[pl.ds(cb, _CWIN)], normc)
        pltpu.async_copy(h.at[rowc.at[0]], msg0, gsem0)

        @pl.loop(0, _CWIN // 2)
        def _pk(k):
            w0 = 2 * k
            w1 = w0 + 1
            pltpu.async_copy(h.at[rowc.at[w1]], msg1, gsem1)
            pltpu.make_async_copy(h.at[rowc.at[w0]], msg0, gsem0).wait()
            scale_scatter(w0, msg0)
            w2 = jnp.minimum(w0 + 2, _CWIN - 1)
            pltpu.async_copy(h.at[rowc.at[w2]], msg0, gsem0)
            pltpu.make_async_copy(h.at[rowc.at[w1]], msg1, gsem1).wait()
            scale_scatter(w1, msg1)

        pltpu.make_async_copy(h.at[rowc.at[0]], msg0, gsem0).wait()

    plsc.subcore_barrier()

    @pl.when(c == 0)
    def _w0():
        pltpu.sync_copy(acc_sp.at[pl.ds(s * _RPT, _RPT)],
                        out0.at[pl.ds(s * _RPT, _RPT)])

    @pl.when(c == 1)
    def _w1():
        pltpu.sync_copy(acc_sp.at[pl.ds(s * _RPT, _RPT)],
                        out1.at[pl.ds(s * _RPT, _RPT)])


@jax.jit
def _prop_sc(h, row2, col2, norm2):
    f = pl.kernel(
        _prop_body,
        out_type=[jax.ShapeDtypeStruct((_NP, _D), jnp.float32),
                  jax.ShapeDtypeStruct((_NP, _D), jnp.float32)],
        mesh=_mesh,
        compiler_params=_sc_params,
        scratch_types=[
            pltpu.VMEM_SHARED((_NP, _D), jnp.float32),
            pltpu.VMEM((_CWIN, _WE), jnp.int32),
            pltpu.VMEM((_CWIN, _WE), jnp.int32),
            pltpu.VMEM((_CWIN, _WE), jnp.float32),
            pltpu.VMEM((_WE, _D), jnp.float32),
            pltpu.VMEM((_WE, _D), jnp.float32),
            pltpu.SemaphoreType.DMA,
            pltpu.SemaphoreType.DMA,
            pltpu.SemaphoreType.DMA,
        ],
    )
    return f(h, row2, col2, norm2)


def _combine_body(a_ref, b_ref, o_ref):
    o_ref[...] = a_ref[...] + b_ref[...]


_combine = pl.pallas_call(
    _combine_body,
    grid=(8,),
    in_specs=[pl.BlockSpec((_NP // 8, _D), lambda i: (i, 0))] * 2,
    out_specs=pl.BlockSpec((_NP // 8, _D), lambda i: (i, 0)),
    out_shape=jax.ShapeDtypeStruct((_NP, _D), jnp.float32),
)


def _layer_body(x_ref, h1_ref, h2_ref, p0_ref, p1_ref, w_ref, b_ref, o_ref):
    acc = jnp.dot(x_ref[...], w_ref[0], preferred_element_type=jnp.float32)
    acc = acc + jnp.dot(h1_ref[...], w_ref[1], preferred_element_type=jnp.float32)
    acc = acc + jnp.dot(h2_ref[...], w_ref[2], preferred_element_type=jnp.float32)
    h3 = p0_ref[...] + p1_ref[...]
    acc = acc + jnp.dot(h3, w_ref[3], preferred_element_type=jnp.float32)
    acc = acc + b_ref[...]
    o_ref[...] = jnp.where(acc > 0, acc, 0.01 * acc)


_layer = pl.pallas_call(
    _layer_body,
    grid=(8,),
    in_specs=[pl.BlockSpec((_NP // 8, _D), lambda i: (i, 0))] * 5
    + [pl.BlockSpec((4, _D, _D), lambda i: (0, 0, 0)),
       pl.BlockSpec((1, _D), lambda i: (0, 0))],
    out_specs=pl.BlockSpec((_NP // 8, _D), lambda i: (i, 0)),
    out_shape=jax.ShapeDtypeStruct((_NP, _D), jnp.float32),
)


def kernel(y, edge_index, edge_attr, W1, b1, W2):
    row = edge_index[0]
    col = edge_index[1]
    pad = _EP - _E
    rowp = jnp.pad(row, (0, pad))
    colp = jnp.pad(col, (0, pad))
    eap = jnp.pad(edge_attr, (0, pad))
    row2 = rowp.reshape(_EW, 128)
    col2 = colp.reshape(_EW, 128)
    ea2 = eap.reshape(_EW, 128)
    rowb = rowp.reshape(_EW64, _WE)
    colb = colp.reshape(_EW64, _WE)
    x = jnp.pad(y, ((0, _NP - _N), (0, 0)))

    norm2 = _norm_sc(row2, col2, ea2)
    normb = norm2.reshape(_EW64, _WE)
    b1r = b1.reshape(1, _D)
    zb = jnp.zeros((1, _D), jnp.float32)

    for W, b in ((W1, b1r), (W2, zb)):
        p10, p11 = _prop_sc(x, rowb, colb, normb)
        h1 = _combine(p10, p11)
        p20, p21 = _prop_sc(h1, rowb, colb, normb)
        h2 = _combine(p20, p21)
        p30, p31 = _prop_sc(h2, rowb, colb, normb)
        x = _layer(x, h1, h2, p30, p31, W, b)
    return x[:_N]


# speedup vs baseline: 4.5740x; 1.0466x over previous
import dataclasses
import functools

import jax
import jax.numpy as jnp
from jax import lax
from jax.experimental import pallas as pl
from jax.experimental.pallas import tpu as pltpu
from jax.experimental.pallas import tpu_sc as plsc

_N = 10000
_E = 320000
_D = 128
_NC = 2
_NS = 16
_NW = _NC * _NS
_L = 16

_NP = 10240
_EP = 327680
_EW = _EP // 128
_WPW = _EW // _NW
_WPT = _EW // _NS
_RPT = _NP // _NS

_mesh = plsc.VectorSubcoreMesh(core_axis_name="c", subcore_axis_name="s")

_sc_params = pltpu.CompilerParams()
if "needs_layout_passes" in pltpu.CompilerParams.__dataclass_fields__:
    _sc_params = dataclasses.replace(_sc_params, needs_layout_passes=False)


def _z16():
    return jnp.zeros((_L,), jnp.float32)


def _norm_body(row2, col2, ea2, norm2,
               deg_sp, dinv_sp, colv, eav, zv, degv, dinvv, dinvf,
               rowc, colc, eac, normc):
    c = lax.axis_index("c")
    s = lax.axis_index("s")
    wid = s * _NC + c

    @pl.loop(0, _RPT // _L)
    def _p1(i):
        zv[pl.ds(i * _L, _L)] = _z16()

    pltpu.sync_copy(zv, deg_sp.at[pl.ds(s * _RPT, _RPT)])
    plsc.subcore_barrier()

    pltpu.sync_copy(col2.at[pl.ds(s * _WPT, _WPT)], colv)
    pltpu.sync_copy(ea2.at[pl.ds(s * _WPT, _WPT)], eav)

    @pl.loop(0, _WPT)
    def _p2(w):
        pltpu.sync_copy(eav.at[w], deg_sp.at[colv.at[w]], add=True)

    plsc.subcore_barrier()

    pltpu.sync_copy(deg_sp.at[pl.ds(s * _RPT, _RPT)], degv)

    @pl.loop(0, _RPT // _L)
    def _p3(i):
        sl = pl.ds(i * _L, _L)
        d = degv[sl]
        bits = plsc.bitcast(d, jnp.int32)
        y = plsc.bitcast(jnp.int32(0x5F3759DF) - (bits >> 1), jnp.float32)
        for _ in range(3):
            y = y * (1.5 - 0.5 * d * y * y)
        dinvv[sl] = jnp.where(d > 0.0, y, 0.0)

    pltpu.sync_copy(dinvv, dinv_sp.at[pl.ds(s * _RPT, _RPT)])
    plsc.subcore_barrier()

    pltpu.sync_copy(dinv_sp, dinvf)
    base = wid * _WPW
    pltpu.sync_copy(row2.at[pl.ds(base, _WPW)], rowc)
    pltpu.sync_copy(col2.at[pl.ds(base, _WPW)], colc)
    pltpu.sync_copy(ea2.at[pl.ds(base, _WPW)], eac)

    @pl.loop(0, _WPW)
    def _p4(w):
        for j in range(8):
            sl = pl.ds(j * _L, _L)
            dr = plsc.load_gather(dinvf, [rowc[w, sl]])
            dc = plsc.load_gather(dinvf, [colc[w, sl]])
            normc[w, sl] = dr * eac[w, sl] * dc

    pltpu.sync_copy(normc, norm2.at[pl.ds(base, _WPW)])


@jax.jit
def _norm_sc(row2, col2, ea2):
    f = pl.kernel(
        _norm_body,
        out_type=jax.ShapeDtypeStruct((_EW, 128), jnp.float32),
        mesh=_mesh,
        compiler_params=_sc_params,
        scratch_types=[
            pltpu.VMEM_SHARED((_NP,), jnp.float32),
            pltpu.VMEM_SHARED((_NP,), jnp.float32),
            pltpu.VMEM((_WPT, 128), jnp.int32),
            pltpu.VMEM((_WPT, 128), jnp.float32),
            pltpu.VMEM((_RPT,), jnp.float32),
            pltpu.VMEM((_RPT,), jnp.float32),
            pltpu.VMEM((_RPT,), jnp.float32),
            pltpu.VMEM((_NP,), jnp.float32),
            pltpu.VMEM((_WPW, 128), jnp.int32),
            pltpu.VMEM((_WPW, 128), jnp.int32),
            pltpu.VMEM((_WPW, 128), jnp.float32),
            pltpu.VMEM((_WPW, 128), jnp.float32),
        ],
    )
    return f(row2, col2, ea2)


_WE = 32
_EWP = _EP // _WE
_W64 = _EWP // _NW
_CWIN = 80
_NB = 4


def _prop_body(h, row2, col2, norm2, out0, out1,
               acc_sp, rowc, colc, normc, msg0, msg1, msg2, msg3,
               gsem0, gsem1, gsem2, gsem3, ssem0, ssem1, ssem2, ssem3, zsem):
    c = lax.axis_index("c")
    s = lax.axis_index("s")
    wid = s * _NC + c
    msg = (msg0, msg1, msg2, msg3)
    gsem = (gsem0, gsem1, gsem2, gsem3)
    ssem = (ssem0, ssem1, ssem2, ssem3)

    @pl.loop(0, _WE)
    def _pz(i):
        for j in range(8):
            msg0[i, pl.ds(j * _L, _L)] = _z16()

    @pl.loop(0, _RPT // _WE)
    def _pz2(i):
        pltpu.async_copy(msg0, acc_sp.at[pl.ds(s * _RPT + i * _WE, _WE)], zsem)

    @pl.loop(0, _RPT // _WE)
    def _pz3(i):
        pltpu.make_async_copy(msg0, acc_sp.at[pl.ds(s * _RPT, _WE)], zsem).wait()

    plsc.subcore_barrier()

    base = wid * _W64

    def scale(w, m):
        @pl.loop(0, _WE)
        def _pe(e):
            nb = plsc.load_gather(
                normc, [jnp.full((_L,), w, jnp.int32),
                        jnp.full((_L,), e, jnp.int32)])
            for j in range(8):
                sl = pl.ds(j * _L, _L)
                m[e, sl] = m[e, sl] * nb

    @pl.loop(0, _W64 // _CWIN)
    def _pc(ci):
        cb = base + ci * _CWIN
        pltpu.sync_copy(row2.at[pl.ds(cb, _CWIN)], rowc)
        pltpu.sync_copy(col2.at[pl.ds(cb, _CWIN)], colc)
        pltpu.sync_copy(norm2.at[pl.ds(cb, _CWIN)], normc)
        for b in range(_NB - 1):
            pltpu.async_copy(h.at[rowc.at[b]], msg[b], gsem[b])

        @pl.loop(0, _CWIN // _NB)
        def _pk(g):
            for b in range(_NB):
                w = g * _NB + b
                pltpu.make_async_copy(h.at[rowc.at[w]], msg[b], gsem[b]).wait()
                scale(w, msg[b])
                pltpu.async_copy(msg[b], acc_sp.at[colc.at[w]], ssem[b],
                                 add=True)
                bp = (b - 1) % _NB
                wn = jnp.minimum(w + _NB - 1, _CWIN - 1)

                def _issue():
                    pltpu.async_copy(h.at[rowc.at[wn]], msg[bp], gsem[bp])

                def _drain_then_issue():
                    pltpu.make_async_copy(
                        msg[bp], acc_sp.at[colc.at[w]], ssem[bp]).wait()
                    _issue()

                if b == 0:
                    @pl.when(g > 0)
                    def _g0():
                        _drain_then_issue()

                    @pl.when(g == 0)
                    def _g0first():
                        _issue()
                else:
                    _drain_then_issue()

        pltpu.make_async_copy(msg[_NB - 1],
                              acc_sp.at[colc.at[0]], ssem[_NB - 1]).wait()
        for b in range(_NB - 1):
            pltpu.make_async_copy(h.at[rowc.at[0]], msg[b], gsem[b]).wait()

    plsc.subcore_barrier()

    @pl.when(c == 0)
    def _w0():
        pltpu.sync_copy(acc_sp.at[pl.ds(s * _RPT, _RPT)],
                        out0.at[pl.ds(s * _RPT, _RPT)])

    @pl.when(c == 1)
    def _w1():
        pltpu.sync_copy(acc_sp.at[pl.ds(s * _RPT, _RPT)],
                        out1.at[pl.ds(s * _RPT, _RPT)])


@jax.jit
def _prop_sc(h, row2, col2, norm2):
    f = pl.kernel(
        _prop_body,
        out_type=[jax.ShapeDtypeStruct((_NP, _D), jnp.float32),
                  jax.ShapeDtypeStruct((_NP, _D), jnp.float32)],
        mesh=_mesh,
        compiler_params=_sc_params,
        scratch_types=[
            pltpu.VMEM_SHARED((_NP, _D), jnp.float32),
            pltpu.VMEM((_CWIN, _WE), jnp.int32),
            pltpu.VMEM((_CWIN, _WE), jnp.int32),
            pltpu.VMEM((_CWIN, _WE), jnp.float32),
            pltpu.VMEM((_WE, _D), jnp.float32),
            pltpu.VMEM((_WE, _D), jnp.float32),
            pltpu.VMEM((_WE, _D), jnp.float32),
            pltpu.VMEM((_WE, _D), jnp.float32),
            pltpu.SemaphoreType.DMA,
            pltpu.SemaphoreType.DMA,
            pltpu.SemaphoreType.DMA,
            pltpu.SemaphoreType.DMA,
            pltpu.SemaphoreType.DMA,
            pltpu.SemaphoreType.DMA,
            pltpu.SemaphoreType.DMA,
            pltpu.SemaphoreType.DMA,
            pltpu.SemaphoreType.DMA,
        ],
    )
    return f(h, row2, col2, norm2)


def _combine_body(a_ref, b_ref, o_ref):
    o_ref[...] = a_ref[...] + b_ref[...]


_combine = pl.pallas_call(
    _combine_body,
    grid=(8,),
    in_specs=[pl.BlockSpec((_NP // 8, _D), lambda i: (i, 0))] * 2,
    out_specs=pl.BlockSpec((_NP // 8, _D), lambda i: (i, 0)),
    out_shape=jax.ShapeDtypeStruct((_NP, _D), jnp.float32),
)


def _layer_body(x_ref, h1_ref, h2_ref, p0_ref, p1_ref, w_ref, b_ref, o_ref):
    acc = jnp.dot(x_ref[...], w_ref[0], preferred_element_type=jnp.float32)
    acc = acc + jnp.dot(h1_ref[...], w_ref[1], preferred_element_type=jnp.float32)
    acc = acc + jnp.dot(h2_ref[...], w_ref[2], preferred_element_type=jnp.float32)
    h3 = p0_ref[...] + p1_ref[...]
    acc = acc + jnp.dot(h3, w_ref[3], preferred_element_type=jnp.float32)
    acc = acc + b_ref[...]
    o_ref[...] = jnp.where(acc > 0, acc, 0.01 * acc)


_layer = pl.pallas_call(
    _layer_body,
    grid=(8,),
    in_specs=[pl.BlockSpec((_NP // 8, _D), lambda i: (i, 0))] * 5
    + [pl.BlockSpec((4, _D, _D), lambda i: (0, 0, 0)),
       pl.BlockSpec((1, _D), lambda i: (0, 0))],
    out_specs=pl.BlockSpec((_NP // 8, _D), lambda i: (i, 0)),
    out_shape=jax.ShapeDtypeStruct((_NP, _D), jnp.float32),
)


def kernel(y, edge_index, edge_attr, W1, b1, W2):
    row = edge_index[0]
    col = edge_index[1]
    pad = _EP - _E
    rowp = jnp.pad(row, (0, pad))
    colp = jnp.pad(col, (0, pad))
    eap = jnp.pad(edge_attr, (0, pad))
    row2 = rowp.reshape(_EW, 128)
    col2 = colp.reshape(_EW, 128)
    ea2 = eap.reshape(_EW, 128)
    rowb = rowp.reshape(_EWP, _WE)
    colb = colp.reshape(_EWP, _WE)
    x = jnp.pad(y, ((0, _NP - _N), (0, 0)))

    norm2 = _norm_sc(row2, col2, ea2)
    normb = norm2.reshape(_EWP, _WE)
    b1r = b1.reshape(1, _D)
    zb = jnp.zeros((1, _D), jnp.float32)

    for W, b in ((W1, b1r), (W2, zb)):
        p10, p11 = _prop_sc(x, rowb, colb, normb)
        h1 = _combine(p10, p11)
        p20, p21 = _prop_sc(h1, rowb, colb, normb)
        h2 = _combine(p20, p21)
        p30, p31 = _prop_sc(h2, rowb, colb, normb)
        x = _layer(x, h1, h2, p30, p31, W, b)
    return x[:_N]


# speedup vs baseline: 5.0103x; 1.0954x over previous
import dataclasses
import functools

import jax
import jax.numpy as jnp
from jax import lax
from jax.experimental import pallas as pl
from jax.experimental.pallas import tpu as pltpu
from jax.experimental.pallas import tpu_sc as plsc

_N = 10000
_E = 320000
_D = 128
_NC = 2
_NS = 16
_NW = _NC * _NS
_L = 16

_NP = 10240
_EP = 327680
_EW = _EP // 128
_WPW = _EW // _NW
_WPT = _EW // _NS
_RPT = _NP // _NS

_mesh = plsc.VectorSubcoreMesh(core_axis_name="c", subcore_axis_name="s")

_sc_params = pltpu.CompilerParams()
if "needs_layout_passes" in pltpu.CompilerParams.__dataclass_fields__:
    _sc_params = dataclasses.replace(_sc_params, needs_layout_passes=False)


def _z16():
    return jnp.zeros((_L,), jnp.float32)


def _norm_body(row2, col2, ea2, norm2,
               deg_sp, dinv_sp, colv, eav, zv, degv, dinvv, dinvf,
               rowc, colc, eac, normc):
    c = lax.axis_index("c")
    s = lax.axis_index("s")
    wid = s * _NC + c

    @pl.loop(0, _RPT // _L)
    def _p1(i):
        zv[pl.ds(i * _L, _L)] = _z16()

    pltpu.sync_copy(zv, deg_sp.at[pl.ds(s * _RPT, _RPT)])
    plsc.subcore_barrier()

    pltpu.sync_copy(col2.at[pl.ds(s * _WPT, _WPT)], colv)
    pltpu.sync_copy(ea2.at[pl.ds(s * _WPT, _WPT)], eav)

    @pl.loop(0, _WPT)
    def _p2(w):
        pltpu.sync_copy(eav.at[w], deg_sp.at[colv.at[w]], add=True)

    plsc.subcore_barrier()

    pltpu.sync_copy(deg_sp.at[pl.ds(s * _RPT, _RPT)], degv)

    @pl.loop(0, _RPT // _L)
    def _p3(i):
        sl = pl.ds(i * _L, _L)
        d = degv[sl]
        bits = plsc.bitcast(d, jnp.int32)
        y = plsc.bitcast(jnp.int32(0x5F3759DF) - (bits >> 1), jnp.float32)
        for _ in range(3):
            y = y * (1.5 - 0.5 * d * y * y)
        dinvv[sl] = jnp.where(d > 0.0, y, 0.0)

    pltpu.sync_copy(dinvv, dinv_sp.at[pl.ds(s * _RPT, _RPT)])
    plsc.subcore_barrier()

    pltpu.sync_copy(dinv_sp, dinvf)
    base = wid * _WPW
    pltpu.sync_copy(row2.at[pl.ds(base, _WPW)], rowc)
    pltpu.sync_copy(col2.at[pl.ds(base, _WPW)], colc)
    pltpu.sync_copy(ea2.at[pl.ds(base, _WPW)], eac)

    @pl.loop(0, _WPW)
    def _p4(w):
        for j in range(8):
            sl = pl.ds(j * _L, _L)
            dr = plsc.load_gather(dinvf, [rowc[w, sl]])
            dc = plsc.load_gather(dinvf, [colc[w, sl]])
            normc[w, sl] = dr * eac[w, sl] * dc

    pltpu.sync_copy(normc, norm2.at[pl.ds(base, _WPW)])


@jax.jit
def _norm_sc(row2, col2, ea2):
    f = pl.kernel(
        _norm_body,
        out_type=jax.ShapeDtypeStruct((_EW, 128), jnp.float32),
        mesh=_mesh,
        compiler_params=_sc_params,
        scratch_types=[
            pltpu.VMEM_SHARED((_NP,), jnp.float32),
            pltpu.VMEM_SHARED((_NP,), jnp.float32),
            pltpu.VMEM((_WPT, 128), jnp.int32),
            pltpu.VMEM((_WPT, 128), jnp.float32),
            pltpu.VMEM((_RPT,), jnp.float32),
            pltpu.VMEM((_RPT,), jnp.float32),
            pltpu.VMEM((_RPT,), jnp.float32),
            pltpu.VMEM((_NP,), jnp.float32),
            pltpu.VMEM((_WPW, 128), jnp.int32),
            pltpu.VMEM((_WPW, 128), jnp.int32),
            pltpu.VMEM((_WPW, 128), jnp.float32),
            pltpu.VMEM((_WPW, 128), jnp.float32),
        ],
    )
    return f(row2, col2, ea2)


_WE = 32
_EWP = _EP // _WE
_W64 = _EWP // _NW
_CWIN = 80
_NB = 4
_T_CH = _EWP // _NS // _CWIN
_F_CH = 6


def _prop_body(h, row2, col2, norm2, out0, out1,
               acc_sp, rowc, colc, normc, msg0, msg1, msg2, msg3,
               gsem0, gsem1, gsem2, gsem3, ssem0, ssem1, ssem2, ssem3, zsem):
    c = lax.axis_index("c")
    s = lax.axis_index("s")
    wid = s * _NC + c
    msg = (msg0, msg1, msg2, msg3)
    gsem = (gsem0, gsem1, gsem2, gsem3)
    ssem = (ssem0, ssem1, ssem2, ssem3)

    @pl.loop(0, _WE)
    def _pz(i):
        for j in range(8):
            msg0[i, pl.ds(j * _L, _L)] = _z16()

    @pl.loop(0, _RPT // _WE)
    def _pz2(i):
        pltpu.async_copy(msg0, acc_sp.at[pl.ds(s * _RPT + i * _WE, _WE)], zsem)

    @pl.loop(0, _RPT // _WE)
    def _pz3(i):
        pltpu.make_async_copy(msg0, acc_sp.at[pl.ds(s * _RPT, _WE)], zsem).wait()

    plsc.subcore_barrier()

    n_chunks = jnp.where(c == 0, _F_CH, _T_CH - _F_CH)
    base = jnp.where(c == 0, s * (_F_CH * _CWIN),
                     _NS * _F_CH * _CWIN + s * ((_T_CH - _F_CH) * _CWIN))

    def scale(w, m):
        @pl.loop(0, _WE)
        def _pe(e):
            nb = plsc.load_gather(
                normc, [jnp.full((_L,), w, jnp.int32),
                        jnp.full((_L,), e, jnp.int32)])
            for j in range(8):
                sl = pl.ds(j * _L, _L)
                m[e, sl] = m[e, sl] * nb

    def _pc(ci, carry):
        cb = base + ci * _CWIN
        pltpu.sync_copy(row2.at[pl.ds(cb, _CWIN)], rowc)
        pltpu.sync_copy(col2.at[pl.ds(cb, _CWIN)], colc)
        pltpu.sync_copy(norm2.at[pl.ds(cb, _CWIN)], normc)
        for b in range(_NB - 1):
            pltpu.async_copy(h.at[rowc.at[b]], msg[b], gsem[b])

        @pl.loop(0, _CWIN // _NB)
        def _pk(g):
            for b in range(_NB):
                w = g * _NB + b
                pltpu.make_async_copy(h.at[rowc.at[w]], msg[b], gsem[b]).wait()
                scale(w, msg[b])
                pltpu.async_copy(msg[b], acc_sp.at[colc.at[w]], ssem[b],
                                 add=True)
                bp = (b - 1) % _NB
                wn = jnp.minimum(w + _NB - 1, _CWIN - 1)

                def _issue():
                    pltpu.async_copy(h.at[rowc.at[wn]], msg[bp], gsem[bp])

                def _drain_then_issue():
                    pltpu.make_async_copy(
                        msg[bp], acc_sp.at[colc.at[w]], ssem[bp]).wait()
                    _issue()

                if b == 0:
                    @pl.when(g > 0)
                    def _g0():
                        _drain_then_issue()

                    @pl.when(g == 0)
                    def _g0first():
                        _issue()
                else:
                    _drain_then_issue()

        pltpu.make_async_copy(msg[_NB - 1],
                              acc_sp.at[colc.at[0]], ssem[_NB - 1]).wait()
        for b in range(_NB - 1):
            pltpu.make_async_copy(h.at[rowc.at[0]], msg[b], gsem[b]).wait()
        return carry

    lax.fori_loop(0, n_chunks, _pc, 0)
    plsc.subcore_barrier()

    @pl.when(c == 0)
    def _w0():
        pltpu.sync_copy(acc_sp.at[pl.ds(s * _RPT, _RPT)],
                        out0.at[pl.ds(s * _RPT, _RPT)])

    @pl.when(c == 1)
    def _w1():
        pltpu.sync_copy(acc_sp.at[pl.ds(s * _RPT, _RPT)],
                        out1.at[pl.ds(s * _RPT, _RPT)])


@jax.jit
def _prop_sc(h, row2, col2, norm2):
    f = pl.kernel(
        _prop_body,
        out_type=[jax.ShapeDtypeStruct((_NP, _D), jnp.float32),
                  jax.ShapeDtypeStruct((_NP, _D), jnp.float32)],
        mesh=_mesh,
        compiler_params=_sc_params,
        scratch_types=[
            pltpu.VMEM_SHARED((_NP, _D), jnp.float32),
            pltpu.VMEM((_CWIN, _WE), jnp.int32),
            pltpu.VMEM((_CWIN, _WE), jnp.int32),
            pltpu.VMEM((_CWIN, _WE), jnp.float32),
            pltpu.VMEM((_WE, _D), jnp.float32),
            pltpu.VMEM((_WE, _D), jnp.float32),
            pltpu.VMEM((_WE, _D), jnp.float32),
            pltpu.VMEM((_WE, _D), jnp.float32),
            pltpu.SemaphoreType.DMA,
            pltpu.SemaphoreType.DMA,
            pltpu.SemaphoreType.DMA,
            pltpu.SemaphoreType.DMA,
            pltpu.SemaphoreType.DMA,
            pltpu.SemaphoreType.DMA,
            pltpu.SemaphoreType.DMA,
            pltpu.SemaphoreType.DMA,
            pltpu.SemaphoreType.DMA,
        ],
    )
    return f(h, row2, col2, norm2)


def _combine_body(a_ref, b_ref, o_ref):
    o_ref[...] = a_ref[...] + b_ref[...]


_combine = pl.pallas_call(
    _combine_body,
    grid=(8,),
    in_specs=[pl.BlockSpec((_NP // 8, _D), lambda i: (i, 0))] * 2,
    out_specs=pl.BlockSpec((_NP // 8, _D), lambda i: (i, 0)),
    out_shape=jax.ShapeDtypeStruct((_NP, _D), jnp.float32),
)


def _layer_body(x_ref, h1_ref, h2_ref, p0_ref, p1_ref, w_ref, b_ref, o_ref):
    acc = jnp.dot(x_ref[...], w_ref[0], preferred_element_type=jnp.float32)
    acc = acc + jnp.dot(h1_ref[...], w_ref[1], preferred_element_type=jnp.float32)
    acc = acc + jnp.dot(h2_ref[...], w_ref[2], preferred_element_type=jnp.float32)
    h3 = p0_ref[...] + p1_ref[...]
    acc = acc + jnp.dot(h3, w_ref[3], preferred_element_type=jnp.float32)
    acc = acc + b_ref[...]
    o_ref[...] = jnp.where(acc > 0, acc, 0.01 * acc)


_layer = pl.pallas_call(
    _layer_body,
    grid=(8,),
    in_specs=[pl.BlockSpec((_NP // 8, _D), lambda i: (i, 0))] * 5
    + [pl.BlockSpec((4, _D, _D), lambda i: (0, 0, 0)),
       pl.BlockSpec((1, _D), lambda i: (0, 0))],
    out_specs=pl.BlockSpec((_NP // 8, _D), lambda i: (i, 0)),
    out_shape=jax.ShapeDtypeStruct((_NP, _D), jnp.float32),
)


def kernel(y, edge_index, edge_attr, W1, b1, W2):
    row = edge_index[0]
    col = edge_index[1]
    pad = _EP - _E
    rowp = jnp.pad(row, (0, pad))
    colp = jnp.pad(col, (0, pad))
    eap = jnp.pad(edge_attr, (0, pad))
    row2 = rowp.reshape(_EW, 128)
    col2 = colp.reshape(_EW, 128)
    ea2 = eap.reshape(_EW, 128)
    rowb = rowp.reshape(_EWP, _WE)
    colb = colp.reshape(_EWP, _WE)
    x = jnp.pad(y, ((0, _NP - _N), (0, 0)))

    norm2 = _norm_sc(row2, col2, ea2)
    normb = norm2.reshape(_EWP, _WE)
    b1r = b1.reshape(1, _D)
    zb = jnp.zeros((1, _D), jnp.float32)

    for W, b in ((W1, b1r), (W2, zb)):
        p10, p11 = _prop_sc(x, rowb, colb, normb)
        h1 = _combine(p10, p11)
        p20, p21 = _prop_sc(h1, rowb, colb, normb)
        h2 = _combine(p20, p21)
        p30, p31 = _prop_sc(h2, rowb, colb, normb)
        x = _layer(x, h1, h2, p30, p31, W, b)
    return x[:_N]


# speedup vs baseline: 5.4727x; 1.0923x over previous
import dataclasses
import functools

import jax
import jax.numpy as jnp
from jax import lax
from jax.experimental import pallas as pl
from jax.experimental.pallas import tpu as pltpu
from jax.experimental.pallas import tpu_sc as plsc

_N = 10000
_E = 320000
_D = 128
_NC = 2
_NS = 16
_NW = _NC * _NS
_L = 16

_NP = 10240
_EP = 327680
_EW = _EP // 128
_WPW = _EW // _NW
_WPT = _EW // _NS
_RPT = _NP // _NS

_mesh = plsc.VectorSubcoreMesh(core_axis_name="c", subcore_axis_name="s")

_sc_params = pltpu.CompilerParams()
if "needs_layout_passes" in pltpu.CompilerParams.__dataclass_fields__:
    _sc_params = dataclasses.replace(_sc_params, needs_layout_passes=False)


def _z16():
    return jnp.zeros((_L,), jnp.float32)


def _norm_body(row2, col2, ea2, norm2,
               deg_sp, dinv_sp, colv, eav, zv, degv, dinvv, dinvf,
               rowc, colc, eac, normc):
    c = lax.axis_index("c")
    s = lax.axis_index("s")
    wid = s * _NC + c

    @pl.loop(0, _RPT // _L)
    def _p1(i):
        zv[pl.ds(i * _L, _L)] = _z16()

    pltpu.sync_copy(zv, deg_sp.at[pl.ds(s * _RPT, _RPT)])
    plsc.subcore_barrier()

    pltpu.sync_copy(col2.at[pl.ds(s * _WPT, _WPT)], colv)
    pltpu.sync_copy(ea2.at[pl.ds(s * _WPT, _WPT)], eav)

    @pl.loop(0, _WPT)
    def _p2(w):
        pltpu.sync_copy(eav.at[w], deg_sp.at[colv.at[w]], add=True)

    plsc.subcore_barrier()

    pltpu.sync_copy(deg_sp.at[pl.ds(s * _RPT, _RPT)], degv)

    @pl.loop(0, _RPT // _L)
    def _p3(i):
        sl = pl.ds(i * _L, _L)
        d = degv[sl]
        bits = plsc.bitcast(d, jnp.int32)
        y = plsc.bitcast(jnp.int32(0x5F3759DF) - (bits >> 1), jnp.float32)
        for _ in range(3):
            y = y * (1.5 - 0.5 * d * y * y)
        dinvv[sl] = jnp.where(d > 0.0, y, 0.0)

    pltpu.sync_copy(dinvv, dinv_sp.at[pl.ds(s * _RPT, _RPT)])
    plsc.subcore_barrier()

    pltpu.sync_copy(dinv_sp, dinvf)
    base = wid * _WPW
    pltpu.sync_copy(row2.at[pl.ds(base, _WPW)], rowc)
    pltpu.sync_copy(col2.at[pl.ds(base, _WPW)], colc)
    pltpu.sync_copy(ea2.at[pl.ds(base, _WPW)], eac)

    @pl.loop(0, _WPW)
    def _p4(w):
        for j in range(8):
            sl = pl.ds(j * _L, _L)
            dr = plsc.load_gather(dinvf, [rowc[w, sl]])
            dc = plsc.load_gather(dinvf, [colc[w, sl]])
            normc[w, sl] = dr * eac[w, sl] * dc

    pltpu.sync_copy(normc, norm2.at[pl.ds(base, _WPW)])


@jax.jit
def _norm_sc(row2, col2, ea2):
    f = pl.kernel(
        _norm_body,
        out_type=jax.ShapeDtypeStruct((_EW, 128), jnp.float32),
        mesh=_mesh,
        compiler_params=_sc_params,
        scratch_types=[
            pltpu.VMEM_SHARED((_NP,), jnp.float32),
            pltpu.VMEM_SHARED((_NP,), jnp.float32),
            pltpu.VMEM((_WPT, 128), jnp.int32),
            pltpu.VMEM((_WPT, 128), jnp.float32),
            pltpu.VMEM((_RPT,), jnp.float32),
            pltpu.VMEM((_RPT,), jnp.float32),
            pltpu.VMEM((_RPT,), jnp.float32),
            pltpu.VMEM((_NP,), jnp.float32),
            pltpu.VMEM((_WPW, 128), jnp.int32),
            pltpu.VMEM((_WPW, 128), jnp.int32),
            pltpu.VMEM((_WPW, 128), jnp.float32),
            pltpu.VMEM((_WPW, 128), jnp.float32),
        ],
    )
    return f(row2, col2, ea2)


_WE = 32
_EWP = _EP // _WE
_W64 = _EWP // _NW
_CWIN = 80
_NB = 4
_T_CH = _EWP // _NS // _CWIN
_F_CH = 7


def _prop_body(h, row2, col2, norm2, out0, out1,
               acc_sp, rowc, colc, normc, msg0, msg1, msg2, msg3,
               gsem0, gsem1, gsem2, gsem3, ssem0, ssem1, ssem2, ssem3, zsem):
    c = lax.axis_index("c")
    s = lax.axis_index("s")
    wid = s * _NC + c
    msg = (msg0, msg1, msg2, msg3)
    gsem = (gsem0, gsem1, gsem2, gsem3)
    ssem = (ssem0, ssem1, ssem2, ssem3)

    @pl.loop(0, _WE)
    def _pz(i):
        for j in range(8):
            msg0[i, pl.ds(j * _L, _L)] = _z16()

    @pl.loop(0, _RPT // _WE)
    def _pz2(i):
        pltpu.async_copy(msg0, acc_sp.at[pl.ds(s * _RPT + i * _WE, _WE)], zsem)

    @pl.loop(0, _RPT // _WE)
    def _pz3(i):
        pltpu.make_async_copy(msg0, acc_sp.at[pl.ds(s * _RPT, _WE)], zsem).wait()

    plsc.subcore_barrier()

    n_chunks = jnp.where(c == 0, _F_CH, _T_CH - _F_CH)
    base = jnp.where(c == 0, s * (_F_CH * _CWIN),
                     _NS * _F_CH * _CWIN + s * ((_T_CH - _F_CH) * _CWIN))

    def scale(w, m):
        @pl.loop(0, _WE)
        def _pe(e):
            nb = plsc.load_gather(
                normc, [jnp.full((_L,), w, jnp.int32),
                        jnp.full((_L,), e, jnp.int32)])
            for j in range(8):
                sl = pl.ds(j * _L, _L)
                m[e, sl] = m[e, sl] * nb

    def _pc(ci, carry):
        cb = base + ci * _CWIN
        pltpu.sync_copy(row2.at[pl.ds(cb, _CWIN)], rowc)
        pltpu.sync_copy(col2.at[pl.ds(cb, _CWIN)], colc)
        pltpu.sync_copy(norm2.at[pl.ds(cb, _CWIN)], normc)
        for b in range(_NB - 1):
            pltpu.async_copy(h.at[rowc.at[b]], msg[b], gsem[b])

        @pl.loop(0, _CWIN // _NB)
        def _pk(g):
            for b in range(_NB):
                w = g * _NB + b
                pltpu.make_async_copy(h.at[rowc.at[w]], msg[b], gsem[b]).wait()
                scale(w, msg[b])
                pltpu.async_copy(msg[b], acc_sp.at[colc.at[w]], ssem[b],
                                 add=True)
                bp = (b - 1) % _NB
                wn = jnp.minimum(w + _NB - 1, _CWIN - 1)

                def _issue():
                    pltpu.async_copy(h.at[rowc.at[wn]], msg[bp], gsem[bp])

                def _drain_then_issue():
                    pltpu.make_async_copy(
                        msg[bp], acc_sp.at[colc.at[w]], ssem[bp]).wait()
                    _issue()

                if b == 0:
                    @pl.when(g > 0)
                    def _g0():
                        _drain_then_issue()

                    @pl.when(g == 0)
                    def _g0first():
                        _issue()
                else:
                    _drain_then_issue()

        pltpu.make_async_copy(msg[_NB - 1],
                              acc_sp.at[colc.at[0]], ssem[_NB - 1]).wait()
        for b in range(_NB - 1):
            pltpu.make_async_copy(h.at[rowc.at[0]], msg[b], gsem[b]).wait()
        return carry

    lax.fori_loop(0, n_chunks, _pc, 0)
    plsc.subcore_barrier()

    @pl.when(c == 0)
    def _w0():
        pltpu.sync_copy(acc_sp.at[pl.ds(s * _RPT, _RPT)],
                        out0.at[pl.ds(s * _RPT, _RPT)])

    @pl.when(c == 1)
    def _w1():
        pltpu.sync_copy(acc_sp.at[pl.ds(s * _RPT, _RPT)],
                        out1.at[pl.ds(s * _RPT, _RPT)])


@jax.jit
def _prop_sc(h, row2, col2, norm2):
    f = pl.kernel(
        _prop_body,
        out_type=[jax.ShapeDtypeStruct((_NP, _D), jnp.float32),
                  jax.ShapeDtypeStruct((_NP, _D), jnp.float32)],
        mesh=_mesh,
        compiler_params=_sc_params,
        scratch_types=[
            pltpu.VMEM_SHARED((_NP, _D), jnp.float32),
            pltpu.VMEM((_CWIN, _WE), jnp.int32),
            pltpu.VMEM((_CWIN, _WE), jnp.int32),
            pltpu.VMEM((_CWIN, _WE), jnp.float32),
            pltpu.VMEM((_WE, _D), jnp.float32),
            pltpu.VMEM((_WE, _D), jnp.float32),
            pltpu.VMEM((_WE, _D), jnp.float32),
            pltpu.VMEM((_WE, _D), jnp.float32),
            pltpu.SemaphoreType.DMA,
            pltpu.SemaphoreType.DMA,
            pltpu.SemaphoreType.DMA,
            pltpu.SemaphoreType.DMA,
            pltpu.SemaphoreType.DMA,
            pltpu.SemaphoreType.DMA,
            pltpu.SemaphoreType.DMA,
            pltpu.SemaphoreType.DMA,
            pltpu.SemaphoreType.DMA,
        ],
    )
    return f(h, row2, col2, norm2)


def _combine_body(a_ref, b_ref, o_ref):
    o_ref[...] = a_ref[...] + b_ref[...]


_combine = pl.pallas_call(
    _combine_body,
    grid=(8,),
    in_specs=[pl.BlockSpec((_NP // 8, _D), lambda i: (i, 0))] * 2,
    out_specs=pl.BlockSpec((_NP // 8, _D), lambda i: (i, 0)),
    out_shape=jax.ShapeDtypeStruct((_NP, _D), jnp.float32),
)


def _layer_body(x_ref, h1_ref, h2_ref, p0_ref, p1_ref, w_ref, b_ref, o_ref):
    acc = jnp.dot(x_ref[...], w_ref[0], preferred_element_type=jnp.float32)
    acc = acc + jnp.dot(h1_ref[...], w_ref[1], preferred_element_type=jnp.float32)
    acc = acc + jnp.dot(h2_ref[...], w_ref[2], preferred_element_type=jnp.float32)
    h3 = p0_ref[...] + p1_ref[...]
    acc = acc + jnp.dot(h3, w_ref[3], preferred_element_type=jnp.float32)
    acc = acc + b_ref[...]
    o_ref[...] = jnp.where(acc > 0, acc, 0.01 * acc)


_layer = pl.pallas_call(
    _layer_body,
    grid=(8,),
    in_specs=[pl.BlockSpec((_NP // 8, _D), lambda i: (i, 0))] * 5
    + [pl.BlockSpec((4, _D, _D), lambda i: (0, 0, 0)),
       pl.BlockSpec((1, _D), lambda i: (0, 0))],
    out_specs=pl.BlockSpec((_NP // 8, _D), lambda i: (i, 0)),
    out_shape=jax.ShapeDtypeStruct((_NP, _D), jnp.float32),
)


def kernel(y, edge_index, edge_attr, W1, b1, W2):
    row = edge_index[0]
    col = edge_index[1]
    pad = _EP - _E
    rowp = jnp.pad(row, (0, pad))
    colp = jnp.pad(col, (0, pad))
    eap = jnp.pad(edge_attr, (0, pad))
    row2 = rowp.reshape(_EW, 128)
    col2 = colp.reshape(_EW, 128)
    ea2 = eap.reshape(_EW, 128)
    rowb = rowp.reshape(_EWP, _WE)
    colb = colp.reshape(_EWP, _WE)
    x = jnp.pad(y, ((0, _NP - _N), (0, 0)))

    norm2 = _norm_sc(row2, col2, ea2)
    normb = norm2.reshape(_EWP, _WE)
    b1r = b1.reshape(1, _D)
    zb = jnp.zeros((1, _D), jnp.float32)

    for W, b in ((W1, b1r), (W2, zb)):
        p10, p11 = _prop_sc(x, rowb, colb, normb)
        h1 = _combine(p10, p11)
        p20, p21 = _prop_sc(h1, rowb, colb, normb)
        h2 = _combine(p20, p21)
        p30, p31 = _prop_sc(h2, rowb, colb, normb)
        x = _layer(x, h1, h2, p30, p31, W, b)
    return x[:_N]
